# baseline (speedup 1.0000x reference)
import math

import jax
import jax.numpy as jnp
from jax.experimental import pallas as pl

_NUM_BASIS = 8
_R_CUT = 5.0
_C1 = math.sqrt(3.0)
_C2 = math.sqrt(15.0)
_C20 = math.sqrt(5.0) / 2.0
_PREF = math.sqrt(2.0 / _R_CUT)

_L = 512
_S = 40


def _edge_kernel(v_ref, len_ref, emb_ref, attr_ref):
    v = v_ref[...]
    x = v[0]
    y = v[1]
    z = v[2]

    r = jnp.sqrt(x * x + y * y + z * z)
    len_ref[...] = r

    inv = 1.0 / jnp.maximum(r, 1e-12)
    ux = x * inv
    uy = y * inv
    uz = z * inv

    attr_ref[...] = jnp.stack(
        [
            jnp.ones_like(ux),
            _C1 * ux,
            _C1 * uy,
            _C1 * uz,
            _C2 * ux * uy,
            _C2 * uy * uz,
            _C20 * (3.0 * uz * uz - 1.0),
            _C2 * ux * uz,
            (_C2 / 2.0) * (ux * ux - uy * uy),
        ],
        axis=0,
    )

    xc = r * (1.0 / _R_CUT)
    x3 = xc * xc * xc
    x6 = x3 * x3
    x7 = x6 * xc
    x8 = x7 * xc
    fc = (1.0 - 28.0 * x6 + 48.0 * x7 - 21.0 * x8) * (xc < 1.0).astype(r.dtype)

    t = (math.pi / _R_CUT) * r
    s1 = jnp.sin(t)
    c2t = 2.0 * jnp.cos(t)
    sins = [s1, c2t * s1]
    for _ in range(_NUM_BASIS - 2):
        sins.append(c2t * sins[-1] - sins[-2])
    scale = (_PREF * inv) * fc
    emb_ref[...] = jnp.stack([s * scale for s in sins], axis=0)


@jax.jit
def kernel(node_feature, edge_vec, edge_index):
    del node_feature, edge_index
    num_edges = edge_vec.shape[0]
    blk = _S * _L
    padded = ((num_edges + blk - 1) // blk) * blk
    rows = padded // _L
    vt = edge_vec.T
    if padded != num_edges:
        vt = jnp.pad(vt, ((0, 0), (0, padded - num_edges)))
    v3 = vt.reshape(3, rows, _L)
    grid = rows // _S

    len2, emb3, attr3 = pl.pallas_call(
        _edge_kernel,
        grid=(grid,),
        in_specs=[pl.BlockSpec((3, _S, _L), lambda i: (0, i, 0))],
        out_specs=[
            pl.BlockSpec((_S, _L), lambda i: (i, 0)),
            pl.BlockSpec((_NUM_BASIS, _S, _L), lambda i: (0, i, 0)),
            pl.BlockSpec((9, _S, _L), lambda i: (0, i, 0)),
        ],
        out_shape=[
            jax.ShapeDtypeStruct((rows, _L), edge_vec.dtype),
            jax.ShapeDtypeStruct((_NUM_BASIS, rows, _L), edge_vec.dtype),
            jax.ShapeDtypeStruct((9, rows, _L), edge_vec.dtype),
        ],
    )(v3)

    edge_length = len2.reshape(padded)[:num_edges]
    edge_embedding = emb3.reshape(_NUM_BASIS, padded)[:, :num_edges].T
    edge_attr = attr3.reshape(9, padded)[:, :num_edges].T
    return edge_length, edge_embedding, edge_attr


# speedup vs baseline: 1.0009x; 1.0009x over previous
import math

import jax
import jax.numpy as jnp
from jax.experimental import pallas as pl

_NUM_BASIS = 8
_R_CUT = 5.0
_C1 = math.sqrt(3.0)
_C2 = math.sqrt(15.0)
_C20 = math.sqrt(5.0) / 2.0
_PREF = math.sqrt(2.0 / _R_CUT)

_L = 512
_S = 40


def _edge_kernel(v_ref, len_ref, emb_ref, attr_ref):
    v = v_ref[...]
    x = v[0]
    y = v[1]
    z = v[2]

    r = jnp.sqrt(x * x + y * y + z * z)
    len_ref[...] = r

    inv = 1.0 / jnp.maximum(r, 1e-12)
    ux = x * inv
    uy = y * inv
    uz = z * inv

    attr_ref[...] = jnp.stack(
        [
            jnp.ones_like(ux),
            _C1 * ux,
            _C1 * uy,
            _C1 * uz,
            _C2 * ux * uy,
            _C2 * uy * uz,
            _C20 * (3.0 * uz * uz - 1.0),
            _C2 * ux * uz,
            (_C2 / 2.0) * (ux * ux - uy * uy),
        ],
        axis=0,
    )

    xc = r * (1.0 / _R_CUT)
    x3 = xc * xc * xc
    x6 = x3 * x3
    x7 = x6 * xc
    x8 = x7 * xc
    fc = (1.0 - 28.0 * x6 + 48.0 * x7 - 21.0 * x8) * (xc < 1.0).astype(r.dtype)

    t = (math.pi / _R_CUT) * r
    s1 = jnp.sin(t)
    c2t = 2.0 * jnp.cos(t)
    sins = [s1, c2t * s1]
    for _ in range(_NUM_BASIS - 2):
        sins.append(c2t * sins[-1] - sins[-2])
    scale = (_PREF * inv) * fc
    emb_ref[...] = jnp.stack([s * scale for s in sins], axis=0)


@jax.jit
def kernel(node_feature, edge_vec, edge_index):
    del node_feature, edge_index
    num_edges = edge_vec.shape[0]
    blk = _S * _L
    padded = ((num_edges + blk - 1) // blk) * blk
    rows = padded // _L
    vt = edge_vec.T
    if padded != num_edges:
        vt = jnp.pad(vt, ((0, 0), (0, padded - num_edges)))
    v3 = vt.reshape(3, rows, _L)
    grid = rows // _S

    len2, emb3, attr3 = pl.pallas_call(
        _edge_kernel,
        grid=(grid,),
        in_specs=[pl.BlockSpec((3, _S, _L), lambda i: (0, i, 0))],
        out_specs=[
            pl.BlockSpec((_S, _L), lambda i: (i, 0)),
            pl.BlockSpec((_NUM_BASIS, _S, _L), lambda i: (0, i, 0)),
            pl.BlockSpec((9, _S, _L), lambda i: (0, i, 0)),
        ],
        out_shape=[
            jax.ShapeDtypeStruct((rows, _L), edge_vec.dtype),
            jax.ShapeDtypeStruct((_NUM_BASIS, rows, _L), edge_vec.dtype),
            jax.ShapeDtypeStruct((9, rows, _L), edge_vec.dtype),
        ],
    )(v3)

    edge_length = len2.reshape(padded)[:num_edges]
    edge_embedding = emb3.reshape(_NUM_BASIS, padded).T[:num_edges]
    edge_attr = attr3.reshape(9, padded).T[:num_edges]
    return edge_length, edge_embedding, edge_attr
